# baseline (speedup 1.0000x reference)
import jax
import jax.numpy as jnp
from jax.experimental import pallas as pl
from jax.experimental.pallas import tpu as pltpu
from jax.experimental.pallas import tpu_sc as plsc

_SC_NUM_CORES = 2
_SC_NUM_SUBCORES = 16
_SC_WORKERS = _SC_NUM_CORES * _SC_NUM_SUBCORES


def _gather_sum_sc(x_flat, W_embedding, batch, ctx, d):
    b_per_w = batch // _SC_WORKERS
    n_idx = ctx * b_per_w

    mesh = plsc.VectorSubcoreMesh(core_axis_name="c", subcore_axis_name="s")

    @pl.kernel(
        out_type=jax.ShapeDtypeStruct((batch, d), jnp.float32),
        mesh=mesh,
        scratch_types=[
            pltpu.VMEM((n_idx,), jnp.int32),
            pltpu.VMEM((n_idx, d), jnp.float32),
            pltpu.VMEM((b_per_w, d), jnp.float32),
            pltpu.SemaphoreType.DMA,
        ],
        compiler_params=pltpu.CompilerParams(use_tc_tiling_on_sc=False),
    )
    def sc_kernel(w_hbm, i_hbm, o_hbm, idx_v, rows_v, s_v, sem):
        wid = jax.lax.axis_index("s") * _SC_NUM_CORES + jax.lax.axis_index("c")
        pltpu.sync_copy(i_hbm.at[pl.ds(wid * n_idx, n_idx)], idx_v)
        pltpu.async_copy(w_hbm.at[idx_v], rows_v, sem).wait()
        for g in range(b_per_w):
            acc = rows_v[ctx * g, :]
            for c in range(1, ctx):
                acc = acc + rows_v[ctx * g + c, :]
            s_v[g, :] = acc
        pltpu.sync_copy(s_v, o_hbm.at[pl.ds(wid * b_per_w, b_per_w)])

    return sc_kernel(W_embedding, x_flat)


def _project_tc(s, U_wT, U_b_row, batch, vocab, d):
    tile_b = 32
    nbuf = 4
    num_tiles = batch // tile_b

    def mm_kernel(s_ref, u_ref, b_ref, o_hbm, slabs, sems):
        def do_slab(i, t):
            row = i * tile_b
            slabs[t] = (
                jax.lax.dot_general(
                    s_ref[pl.ds(row, tile_b), :].astype(jnp.bfloat16),
                    u_ref[...].astype(jnp.bfloat16),
                    (((1,), (0,)), ((), ())),
                    preferred_element_type=jnp.float32,
                )
                + b_ref[...]
            ).astype(jnp.bfloat16)
            pltpu.make_async_copy(
                slabs.at[t], o_hbm.at[pl.ds(row, tile_b), :], sems.at[t]
            ).start()

        for t in range(nbuf):
            do_slab(t, t)

        def outer(j, carry):
            for t in range(nbuf):
                i = j * nbuf + t
                pltpu.make_async_copy(
                    slabs.at[t], o_hbm.at[pl.ds(0, tile_b), :], sems.at[t]
                ).wait()
                do_slab(i, t)
            return carry

        jax.lax.fori_loop(1, num_tiles // nbuf, outer, 0)

        rem = num_tiles % nbuf
        for t in range(rem):
            i = (num_tiles // nbuf) * nbuf + t
            pltpu.make_async_copy(
                slabs.at[t], o_hbm.at[pl.ds(0, tile_b), :], sems.at[t]
            ).wait()
            do_slab(i, t)
        for t in range(nbuf):
            pltpu.make_async_copy(
                slabs.at[t], o_hbm.at[pl.ds(0, tile_b), :], sems.at[t]
            ).wait()

    return pl.pallas_call(
        mm_kernel,
        in_specs=[
            pl.BlockSpec(memory_space=pltpu.VMEM),
            pl.BlockSpec(memory_space=pltpu.VMEM),
            pl.BlockSpec(memory_space=pltpu.VMEM),
        ],
        out_specs=pl.BlockSpec(memory_space=pl.ANY),
        out_shape=jax.ShapeDtypeStruct((batch, vocab), jnp.bfloat16),
        scratch_shapes=[
            pltpu.VMEM((nbuf, tile_b, vocab), jnp.bfloat16),
            pltpu.SemaphoreType.DMA((nbuf,)),
        ],
    )(s, U_wT, U_b_row)


def kernel(x, W_embedding, U_w, U_b):
    batch, ctx = x.shape
    vocab, d = W_embedding.shape
    x_flat = x.reshape(batch * ctx)
    s = _gather_sum_sc(x_flat, W_embedding, batch, ctx, d)
    mm = _project_tc(s, U_w.T, U_b.reshape(1, vocab), batch, vocab, d)
    return mm.astype(jnp.float32)
